# baseline (speedup 1.0000x reference)
import jax
import jax.numpy as jnp
from jax.experimental import pallas as pl
from jax.experimental.pallas import tpu as pltpu

_D = 384
_QK = 384
_M = 12
_CH = 32
_NW = 7
_TOPK = 4
_H = 56
_W = 56
_P2 = _NW * _NW
_W2 = 64
_B = 4
_HID = 4 * _D
_SCALE = _QK ** (-0.5)
_EPS = 1e-5

import numpy as _np
_VMASK_NP = (_np.arange(_D)[None, :] // _CH == _np.arange(_M)[:, None]).astype(_np.float32)
_QMASK_NP = _np.repeat(_VMASK_NP, _W2, axis=0)


def _zero_pad2d(a, p):
    h, w, c = a.shape
    zc = jnp.zeros((h, p, c), jnp.float32)
    a = jnp.concatenate([zc, a, zc], axis=1)
    zr = jnp.zeros((p, w + 2 * p, c), jnp.float32)
    return jnp.concatenate([zr, a, zr], axis=0)


def _pre_body(x_ref, posw_ref, posb_ref, ln1w_ref, ln1b_ref, qkvw_ref, qkvb_ref,
              x1_ref, qkv_ref, ridx_ref):
    x = x_ref[0]
    pad = _zero_pad2d(x, 1)
    cols = [pad[:, kw:kw + _W, :] for kw in range(3)]
    acc = posb_ref[...][None, None, :]
    for kh in range(3):
        for kw in range(3):
            acc = acc + cols[kw][kh:kh + _H] * posw_ref[kh * 3 + kw][None, None, :]
    x1 = x + acc
    x1_ref[0] = x1
    mu = jnp.mean(x1, axis=-1, keepdims=True)
    xc = x1 - mu
    var = jnp.mean(xc * xc, axis=-1, keepdims=True)
    xn = xc * jax.lax.rsqrt(var + _EPS) * ln1w_ref[...] + ln1b_ref[...]
    qkv = jnp.dot(xn.reshape(_H * _W, _D), qkvw_ref[...],
                  preferred_element_type=jnp.float32) + qkvb_ref[...]
    qkv_ref[0] = qkv.reshape(_H, _W, 2 * _QK + _D)
    q5 = qkv.reshape(_NW, 8, _NW, 8, 2 * _QK + _D)
    qwin = jnp.mean(q5[..., :_QK], axis=(1, 3)).reshape(_P2, _QK)
    kwin = jnp.mean(q5[..., _QK:2 * _QK], axis=(1, 3)).reshape(_P2, _QK)
    logits = jax.lax.dot_general(qwin * _SCALE, kwin, (((1,), (1,)), ((), ())),
                                 preferred_element_type=jnp.float32)
    iota = jax.lax.broadcasted_iota(jnp.int32, (_P2, _P2), 1)
    lg = logits
    cols = []
    for _ in range(_TOPK):
        m = jnp.max(lg, axis=1, keepdims=True)
        cand = jnp.where(lg == m, iota, _P2 + 1)
        sel = jnp.min(cand, axis=1)
        cols.append(sel)
        lg = jnp.where(iota == sel[:, None], -jnp.inf, lg)
    ridx_ref[0] = jnp.stack(cols, axis=1)


def _attn_body(s_ref, qkv_ref, qmask_ref, vmask_ref, out_ref):
    n = pl.program_id(0)
    jw = pl.program_id(1)
    qmask = qmask_ref[...]
    vmask = vmask_ref[...]
    for iw in range(_NW):
        q = qkv_ref[0, pl.ds(jw * 8, 8), pl.ds(iw * 8, 8), pl.ds(0, _QK)]
        q = q.reshape(_W2, _QK) * _SCALE
        base = (n * _P2 + jw * _NW + iw) * _TOPK
        ks = []
        vs = []
        for t in range(_TOPK):
            it = s_ref[base + t]
            jj = it // _NW
            ii = it % _NW
            kvt = qkv_ref[0, pl.ds(jj * 8, 8), pl.ds(ii * 8, 8), pl.ds(_QK, _QK + _D)]
            kvt = kvt.reshape(_W2, _QK + _D)
            ks.append(kvt[:, :_QK])
            vs.append(kvt[:, _QK:])
        k = jnp.concatenate(ks, axis=0)
        v = jnp.concatenate(vs, axis=0)
        qbd = jnp.concatenate([q] * _M, axis=0) * qmask
        lg = jax.lax.dot_general(qbd, k, (((1,), (1,)), ((), ())),
                                 preferred_element_type=jnp.float32)
        m = jnp.max(lg, axis=1, keepdims=True)
        e = jnp.exp(lg - m)
        a = e / jnp.sum(e, axis=1, keepdims=True)
        os = jax.lax.dot_general(a, v, (((1,), (0,)), ((), ())),
                                 preferred_element_type=jnp.float32)
        out = jnp.sum(os.reshape(_M, _W2, _D) * vmask[:, None, :], axis=0)
        out_ref[0, :, iw * 8:(iw + 1) * 8, :] = out.reshape(8, 8, _D)


def _post1_body(v_ref, attn_ref, x1_ref, lepew_ref, lepeb_ref, wow_ref, wob_ref,
                x2_ref):
    v = v_ref[0]
    pad = _zero_pad2d(v, 2)
    cols = [pad[:, kw:kw + _W, :] for kw in range(5)]
    acc = lepeb_ref[...][None, None, :]
    for kh in range(5):
        for kw in range(5):
            acc = acc + cols[kw][kh:kh + _H] * lepew_ref[kh * 5 + kw][None, None, :]
    y = attn_ref[0] + acc
    yw = jnp.dot(y.reshape(_H * _W, _D), wow_ref[...],
                 preferred_element_type=jnp.float32) + wob_ref[...]
    x2_ref[0] = x1_ref[0] + yw.reshape(_H, _W, _D)


def _post2_body(x2_ref, ln2w_ref, ln2b_ref, w1_ref, b1_ref, w2_ref, b2_ref, out_ref):
    x2 = x2_ref[...]
    mu = jnp.mean(x2, axis=-1, keepdims=True)
    xc = x2 - mu
    var = jnp.mean(xc * xc, axis=-1, keepdims=True)
    h = xc * jax.lax.rsqrt(var + _EPS) * ln2w_ref[...] + ln2b_ref[...]
    h1 = jnp.dot(h, w1_ref[...], preferred_element_type=jnp.float32) + b1_ref[...]
    g = 0.5 * h1 * (1.0 + jax.lax.erf(h1 * (2.0 ** -0.5)))
    h2 = jnp.dot(g, w2_ref[...], preferred_element_type=jnp.float32) + b2_ref[...]
    out_ref[...] = x2 + h2


def kernel(x, pos_w, pos_b, ln1_w, ln1_b, qkv_w, qkv_b, lepe_w, lepe_b,
           wo_w, wo_b, ln2_w, ln2_b, mlp_w1, mlp_b1, mlp_w2, mlp_b2):
    f32 = jnp.float32
    xh = x.transpose(0, 2, 3, 1)
    posw = pos_w.reshape(_D, 3, 3).transpose(1, 2, 0).reshape(9, _D)
    lepew = lepe_w.reshape(_D, 5, 5).transpose(1, 2, 0).reshape(25, _D)
    qkvwt = qkv_w.T
    wowt = wo_w.T
    w1t = mlp_w1.T
    w2t = mlp_w2.T

    full = lambda s: pl.BlockSpec(s, lambda n: tuple(0 for _ in s))
    x1, qkv, ridx = pl.pallas_call(
        _pre_body,
        grid=(_B,),
        in_specs=[
            pl.BlockSpec((1, _H, _W, _D), lambda n: (n, 0, 0, 0)),
            full((9, _D)), full((_D,)), full((_D,)), full((_D,)),
            full((_D, 2 * _QK + _D)), full((2 * _QK + _D,)),
        ],
        out_specs=[
            pl.BlockSpec((1, _H, _W, _D), lambda n: (n, 0, 0, 0)),
            pl.BlockSpec((1, _H, _W, 2 * _QK + _D), lambda n: (n, 0, 0, 0)),
            pl.BlockSpec((1, _P2, _TOPK), lambda n: (n, 0, 0)),
        ],
        out_shape=[
            jax.ShapeDtypeStruct((_B, _H, _W, _D), f32),
            jax.ShapeDtypeStruct((_B, _H, _W, 2 * _QK + _D), f32),
            jax.ShapeDtypeStruct((_B, _P2, _TOPK), jnp.int32),
        ],
    )(xh, posw, pos_b, ln1_w, ln1_b, qkvwt, qkv_b)

    attn = pl.pallas_call(
        _attn_body,
        grid_spec=pltpu.PrefetchScalarGridSpec(
            num_scalar_prefetch=1,
            grid=(_B, _NW),
            in_specs=[
                pl.BlockSpec((1, _H, _W, 2 * _QK + _D), lambda n, j, s: (n, 0, 0, 0)),
                pl.BlockSpec((_M * _W2, _D), lambda n, j, s: (0, 0)),
                pl.BlockSpec((_M, _D), lambda n, j, s: (0, 0)),
            ],
            out_specs=pl.BlockSpec((1, 8, _W, _D), lambda n, j, s: (n, j, 0, 0)),
        ),
        out_shape=jax.ShapeDtypeStruct((_B, _H, _W, _D), f32),
    )(ridx.reshape(-1), qkv, jnp.asarray(_QMASK_NP), jnp.asarray(_VMASK_NP))

    v_img = qkv[..., _QK + _QK:]
    x2 = pl.pallas_call(
        _post1_body,
        grid=(_B,),
        in_specs=[
            pl.BlockSpec((1, _H, _W, _D), lambda n: (n, 0, 0, 0)),
            pl.BlockSpec((1, _H, _W, _D), lambda n: (n, 0, 0, 0)),
            pl.BlockSpec((1, _H, _W, _D), lambda n: (n, 0, 0, 0)),
            full((25, _D)), full((_D,)), full((_D, _D)), full((_D,)),
        ],
        out_specs=[pl.BlockSpec((1, _H, _W, _D), lambda n: (n, 0, 0, 0))],
        out_shape=[jax.ShapeDtypeStruct((_B, _H, _W, _D), f32)],
    )(v_img, attn, x1, lepew, lepe_b, wowt, wo_b)[0]

    rows = _B * _H * _W
    tile = rows // 8
    out = pl.pallas_call(
        _post2_body,
        grid=(8,),
        in_specs=[
            pl.BlockSpec((tile, _D), lambda r: (r, 0)),
            full((_D,)), full((_D,)),
            full((_D, _HID)), full((_HID,)), full((_HID, _D)), full((_D,)),
        ],
        out_specs=pl.BlockSpec((tile, _D), lambda r: (r, 0)),
        out_shape=jax.ShapeDtypeStruct((rows, _D), f32),
    )(x2.reshape(rows, _D), ln2_w, ln2_b, w1t, mlp_b1, w2t, mlp_b2)

    return out.reshape(_B, _H, _W, _D).transpose(0, 3, 1, 2)


# speedup vs baseline: 1.0384x; 1.0384x over previous
import jax
import jax.numpy as jnp
from jax.experimental import pallas as pl
from jax.experimental.pallas import tpu as pltpu

_D = 384
_QK = 384
_M = 12
_CH = 32
_NW = 7
_TOPK = 4
_H = 56
_W = 56
_P2 = _NW * _NW
_W2 = 64
_B = 4
_HID = 4 * _D
_SCALE = _QK ** (-0.5)
_EPS = 1e-5


def _zero_pad2d(a, p):
    h, w, c = a.shape
    zc = jnp.zeros((h, p, c), jnp.float32)
    a = jnp.concatenate([zc, a, zc], axis=1)
    zr = jnp.zeros((p, w + 2 * p, c), jnp.float32)
    return jnp.concatenate([zr, a, zr], axis=0)


def _pre_body(x_ref, posw_ref, posb_ref, ln1w_ref, ln1b_ref, qkvw_ref, qkvb_ref,
              x1_ref, qkv_ref, ridx_ref):
    x = x_ref[0]
    pad = _zero_pad2d(x, 1)
    cols = [pad[:, kw:kw + _W, :] for kw in range(3)]
    acc = posb_ref[...][None, None, :]
    for kh in range(3):
        for kw in range(3):
            acc = acc + cols[kw][kh:kh + _H] * posw_ref[kh * 3 + kw][None, None, :]
    x1 = x + acc
    x1_ref[0] = x1
    mu = jnp.mean(x1, axis=-1, keepdims=True)
    xc = x1 - mu
    var = jnp.mean(xc * xc, axis=-1, keepdims=True)
    xn = xc * jax.lax.rsqrt(var + _EPS) * ln1w_ref[...] + ln1b_ref[...]
    qkv = jnp.dot(xn.reshape(_H * _W, _D), qkvw_ref[...],
                  preferred_element_type=jnp.float32) + qkvb_ref[...]
    qkv_ref[0] = qkv.reshape(_H, _W, 2 * _QK + _D)
    q5 = qkv.reshape(_NW, 8, _NW, 8, 2 * _QK + _D)
    qwin = jnp.mean(q5[..., :_QK], axis=(1, 3)).reshape(_P2, _QK)
    kwin = jnp.mean(q5[..., _QK:2 * _QK], axis=(1, 3)).reshape(_P2, _QK)
    logits = jax.lax.dot_general(qwin * _SCALE, kwin, (((1,), (1,)), ((), ())),
                                 preferred_element_type=jnp.float32)
    iota = jax.lax.broadcasted_iota(jnp.int32, (_P2, _P2), 1)
    lg = logits
    cols = []
    for _ in range(_TOPK):
        m = jnp.max(lg, axis=1, keepdims=True)
        cand = jnp.where(lg == m, iota, _P2 + 1)
        sel = jnp.min(cand, axis=1)
        cols.append(sel)
        lg = jnp.where(iota == sel[:, None], -jnp.inf, lg)
    ridx_ref[0] = jnp.stack(cols, axis=1)


def _attn_body(s_ref, qkv_ref, out_ref, qbd_ref):
    n = pl.program_id(0)
    jw = pl.program_id(1)

    @pl.when(jnp.logical_and(n == 0, jw == 0))
    def _init():
        qbd_ref[...] = jnp.zeros((_M * _W2, _QK), jnp.float32)

    for iw in range(_NW):
        q = qkv_ref[0, pl.ds(jw * 8, 8), pl.ds(iw * 8, 8), pl.ds(0, _QK)]
        q = q.reshape(_W2, _QK) * _SCALE
        base = (n * _P2 + jw * _NW + iw) * _TOPK
        ks = []
        vs = []
        for t in range(_TOPK):
            it = s_ref[base + t]
            jj = it // _NW
            ii = it % _NW
            kvt = qkv_ref[0, pl.ds(jj * 8, 8), pl.ds(ii * 8, 8), pl.ds(_QK, _QK + _D)]
            kvt = kvt.reshape(_W2, _QK + _D)
            ks.append(kvt[:, :_QK])
            vs.append(kvt[:, _QK:])
        k = jnp.concatenate(ks, axis=0)
        v = jnp.concatenate(vs, axis=0)
        for h in range(_M):
            qbd_ref[h * _W2:(h + 1) * _W2, h * _CH:(h + 1) * _CH] = q[:, h * _CH:(h + 1) * _CH]
        lg = jax.lax.dot_general(qbd_ref[...], k, (((1,), (1,)), ((), ())),
                                 preferred_element_type=jnp.float32)
        m = jnp.max(lg, axis=1, keepdims=True)
        e = jnp.exp(lg - m)
        a = e / jnp.sum(e, axis=1, keepdims=True)
        os = jax.lax.dot_general(a, v, (((1,), (0,)), ((), ())),
                                 preferred_element_type=jnp.float32)
        out = jnp.concatenate(
            [os[h * _W2:(h + 1) * _W2, h * _CH:(h + 1) * _CH] for h in range(_M)], axis=1)
        out_ref[0, :, iw * 8:(iw + 1) * 8, :] = out.reshape(8, 8, _D)


def _post1_body(v_ref, attn_ref, x1_ref, lepew_ref, lepeb_ref, wow_ref, wob_ref,
                x2_ref):
    v = v_ref[0]
    pad = _zero_pad2d(v, 2)
    cols = [pad[:, kw:kw + _W, :] for kw in range(5)]
    acc = lepeb_ref[...][None, None, :]
    for kh in range(5):
        for kw in range(5):
            acc = acc + cols[kw][kh:kh + _H] * lepew_ref[kh * 5 + kw][None, None, :]
    y = attn_ref[0] + acc
    yw = jnp.dot(y.reshape(_H * _W, _D), wow_ref[...],
                 preferred_element_type=jnp.float32) + wob_ref[...]
    x2_ref[0] = x1_ref[0] + yw.reshape(_H, _W, _D)


def _post2_body(x2_ref, ln2w_ref, ln2b_ref, w1_ref, b1_ref, w2_ref, b2_ref, out_ref):
    x2 = x2_ref[...]
    mu = jnp.mean(x2, axis=-1, keepdims=True)
    xc = x2 - mu
    var = jnp.mean(xc * xc, axis=-1, keepdims=True)
    h = xc * jax.lax.rsqrt(var + _EPS) * ln2w_ref[...] + ln2b_ref[...]
    h1 = jnp.dot(h, w1_ref[...], preferred_element_type=jnp.float32) + b1_ref[...]
    g = 0.5 * h1 * (1.0 + jax.lax.erf(h1 * (2.0 ** -0.5)))
    h2 = jnp.dot(g, w2_ref[...], preferred_element_type=jnp.float32) + b2_ref[...]
    out_ref[...] = x2 + h2


def kernel(x, pos_w, pos_b, ln1_w, ln1_b, qkv_w, qkv_b, lepe_w, lepe_b,
           wo_w, wo_b, ln2_w, ln2_b, mlp_w1, mlp_b1, mlp_w2, mlp_b2):
    f32 = jnp.float32
    xh = x.transpose(0, 2, 3, 1)
    posw = pos_w.reshape(_D, 3, 3).transpose(1, 2, 0).reshape(9, _D)
    lepew = lepe_w.reshape(_D, 5, 5).transpose(1, 2, 0).reshape(25, _D)
    qkvwt = qkv_w.T
    wowt = wo_w.T
    w1t = mlp_w1.T
    w2t = mlp_w2.T

    full = lambda s: pl.BlockSpec(s, lambda n: tuple(0 for _ in s))
    x1, qkv, ridx = pl.pallas_call(
        _pre_body,
        grid=(_B,),
        in_specs=[
            pl.BlockSpec((1, _H, _W, _D), lambda n: (n, 0, 0, 0)),
            full((9, _D)), full((_D,)), full((_D,)), full((_D,)),
            full((_D, 2 * _QK + _D)), full((2 * _QK + _D,)),
        ],
        out_specs=[
            pl.BlockSpec((1, _H, _W, _D), lambda n: (n, 0, 0, 0)),
            pl.BlockSpec((1, _H, _W, 2 * _QK + _D), lambda n: (n, 0, 0, 0)),
            pl.BlockSpec((1, _P2, _TOPK), lambda n: (n, 0, 0)),
        ],
        out_shape=[
            jax.ShapeDtypeStruct((_B, _H, _W, _D), f32),
            jax.ShapeDtypeStruct((_B, _H, _W, 2 * _QK + _D), f32),
            jax.ShapeDtypeStruct((_B, _P2, _TOPK), jnp.int32),
        ],
    )(xh, posw, pos_b, ln1_w, ln1_b, qkvwt, qkv_b)

    attn = pl.pallas_call(
        _attn_body,
        grid_spec=pltpu.PrefetchScalarGridSpec(
            num_scalar_prefetch=1,
            grid=(_B, _NW),
            in_specs=[
                pl.BlockSpec((1, _H, _W, 2 * _QK + _D), lambda n, j, s: (n, 0, 0, 0)),
            ],
            out_specs=pl.BlockSpec((1, 8, _W, _D), lambda n, j, s: (n, j, 0, 0)),
            scratch_shapes=[pltpu.VMEM((_M * _W2, _QK), jnp.float32)],
        ),
        out_shape=jax.ShapeDtypeStruct((_B, _H, _W, _D), f32),
    )(ridx.reshape(-1), qkv)

    x2 = pl.pallas_call(
        _post1_body,
        grid=(_B,),
        in_specs=[
            pl.BlockSpec((1, _H, _W, _D), lambda n: (n, 0, 0, 2)),
            pl.BlockSpec((1, _H, _W, _D), lambda n: (n, 0, 0, 0)),
            pl.BlockSpec((1, _H, _W, _D), lambda n: (n, 0, 0, 0)),
            full((25, _D)), full((_D,)), full((_D, _D)), full((_D,)),
        ],
        out_specs=[pl.BlockSpec((1, _H, _W, _D), lambda n: (n, 0, 0, 0))],
        out_shape=[jax.ShapeDtypeStruct((_B, _H, _W, _D), f32)],
    )(qkv, attn, x1, lepew, lepe_b, wowt, wo_b)[0]

    rows = _B * _H * _W
    tile = rows // 8
    out = pl.pallas_call(
        _post2_body,
        grid=(8,),
        in_specs=[
            pl.BlockSpec((tile, _D), lambda r: (r, 0)),
            full((_D,)), full((_D,)),
            full((_D, _HID)), full((_HID,)), full((_HID, _D)), full((_D,)),
        ],
        out_specs=pl.BlockSpec((tile, _D), lambda r: (r, 0)),
        out_shape=jax.ShapeDtypeStruct((rows, _D), f32),
    )(x2.reshape(rows, _D), ln2_w, ln2_b, w1t, mlp_b1, w2t, mlp_b2)

    return out.reshape(_B, _H, _W, _D).transpose(0, 3, 1, 2)


# speedup vs baseline: 1.0756x; 1.0358x over previous
import jax
import jax.numpy as jnp
from jax.experimental import pallas as pl
from jax.experimental.pallas import tpu as pltpu

_D = 384
_QK = 384
_M = 12
_CH = 32
_NW = 7
_TOPK = 4
_H = 56
_W = 56
_P2 = _NW * _NW
_W2 = 64
_B = 4
_HID = 4 * _D
_SCALE = _QK ** (-0.5)
_EPS = 1e-5


def _zero_pad2d(a, p):
    h, w, c = a.shape
    zc = jnp.zeros((h, p, c), jnp.float32)
    a = jnp.concatenate([zc, a, zc], axis=1)
    zr = jnp.zeros((p, w + 2 * p, c), jnp.float32)
    return jnp.concatenate([zr, a, zr], axis=0)


def _pre_body(x_ref, posw_ref, posb_ref, ln1w_ref, ln1b_ref, qkvw_ref, qkvb_ref,
              x1_ref, qkv_ref, ridx_ref):
    x = x_ref[0]
    pad = _zero_pad2d(x, 1)
    cols = [pad[:, kw:kw + _W, :] for kw in range(3)]
    acc = posb_ref[...][None, None, :]
    for kh in range(3):
        for kw in range(3):
            acc = acc + cols[kw][kh:kh + _H] * posw_ref[kh * 3 + kw][None, None, :]
    x1 = x + acc
    x1_ref[0] = x1
    mu = jnp.mean(x1, axis=-1, keepdims=True)
    xc = x1 - mu
    var = jnp.mean(xc * xc, axis=-1, keepdims=True)
    xn = xc * jax.lax.rsqrt(var + _EPS) * ln1w_ref[...] + ln1b_ref[...]
    qkv = jnp.dot(xn.reshape(_H * _W, _D).astype(jnp.bfloat16), qkvw_ref[...],
                  preferred_element_type=jnp.float32) + qkvb_ref[...]
    qkv_ref[0] = qkv.reshape(_H, _W, 2 * _QK + _D)
    q5 = qkv.reshape(_NW, 8, _NW, 8, 2 * _QK + _D)
    qwin = jnp.mean(q5[..., :_QK], axis=(1, 3)).reshape(_P2, _QK)
    kwin = jnp.mean(q5[..., _QK:2 * _QK], axis=(1, 3)).reshape(_P2, _QK)
    logits = jax.lax.dot_general(qwin * _SCALE, kwin, (((1,), (1,)), ((), ())),
                                 preferred_element_type=jnp.float32)
    iota = jax.lax.broadcasted_iota(jnp.int32, (_P2, _P2), 1)
    lg = logits
    cols = []
    for _ in range(_TOPK):
        m = jnp.max(lg, axis=1, keepdims=True)
        cand = jnp.where(lg == m, iota, _P2 + 1)
        sel = jnp.min(cand, axis=1)
        cols.append(sel)
        lg = jnp.where(iota == sel[:, None], -jnp.inf, lg)
    ridx_ref[0] = jnp.stack(cols, axis=1)


def _attn_body(s_ref, qkv_ref, out_ref, qbd_ref):
    n = pl.program_id(0)
    jw = pl.program_id(1)

    @pl.when(jnp.logical_and(n == 0, jw == 0))
    def _init():
        qbd_ref[...] = jnp.zeros((_M * _W2, _QK), jnp.bfloat16)

    for iw in range(_NW):
        q = qkv_ref[0, pl.ds(jw * 8, 8), pl.ds(iw * 8, 8), pl.ds(0, _QK)]
        q = q.reshape(_W2, _QK) * _SCALE
        base = (n * _P2 + jw * _NW + iw) * _TOPK
        ks = []
        vs = []
        for t in range(_TOPK):
            it = s_ref[base + t]
            jj = it // _NW
            ii = it % _NW
            kvt = qkv_ref[0, pl.ds(jj * 8, 8), pl.ds(ii * 8, 8), pl.ds(_QK, _QK + _D)]
            kvt = kvt.reshape(_W2, _QK + _D)
            ks.append(kvt[:, :_QK])
            vs.append(kvt[:, _QK:])
        k = jnp.concatenate(ks, axis=0).astype(jnp.bfloat16)
        v = jnp.concatenate(vs, axis=0).astype(jnp.bfloat16)
        qb = q.astype(jnp.bfloat16)
        for h in range(_M):
            qbd_ref[h * _W2:(h + 1) * _W2, h * _CH:(h + 1) * _CH] = qb[:, h * _CH:(h + 1) * _CH]
        lg = jax.lax.dot_general(qbd_ref[...], k, (((1,), (1,)), ((), ())),
                                 preferred_element_type=jnp.float32)
        m = jnp.max(lg, axis=1, keepdims=True)
        e = jnp.exp(lg - m)
        a = (e / jnp.sum(e, axis=1, keepdims=True)).astype(jnp.bfloat16)
        os = jax.lax.dot_general(a, v, (((1,), (0,)), ((), ())),
                                 preferred_element_type=jnp.float32)
        out = jnp.concatenate(
            [os[h * _W2:(h + 1) * _W2, h * _CH:(h + 1) * _CH] for h in range(_M)], axis=1)
        out_ref[0, :, iw * 8:(iw + 1) * 8, :] = out.reshape(8, 8, _D)


def _post1_body(v_ref, attn_ref, x1_ref, lepew_ref, lepeb_ref, wow_ref, wob_ref,
                x2_ref):
    v = v_ref[0]
    pad = _zero_pad2d(v, 2)
    cols = [pad[:, kw:kw + _W, :] for kw in range(5)]
    acc = lepeb_ref[...][None, None, :]
    for kh in range(5):
        for kw in range(5):
            acc = acc + cols[kw][kh:kh + _H] * lepew_ref[kh * 5 + kw][None, None, :]
    y = attn_ref[0] + acc
    yw = jnp.dot(y.reshape(_H * _W, _D).astype(jnp.bfloat16), wow_ref[...],
                 preferred_element_type=jnp.float32) + wob_ref[...]
    x2_ref[0] = x1_ref[0] + yw.reshape(_H, _W, _D)


def _post2_body(x2_ref, ln2w_ref, ln2b_ref, w1_ref, b1_ref, w2_ref, b2_ref, out_ref):
    x2 = x2_ref[...]
    mu = jnp.mean(x2, axis=-1, keepdims=True)
    xc = x2 - mu
    var = jnp.mean(xc * xc, axis=-1, keepdims=True)
    h = xc * jax.lax.rsqrt(var + _EPS) * ln2w_ref[...] + ln2b_ref[...]
    h1 = jnp.dot(h.astype(jnp.bfloat16), w1_ref[...],
                 preferred_element_type=jnp.float32) + b1_ref[...]
    g = 0.5 * h1 * (1.0 + jax.lax.erf(h1 * (2.0 ** -0.5)))
    h2 = jnp.dot(g.astype(jnp.bfloat16), w2_ref[...],
                 preferred_element_type=jnp.float32) + b2_ref[...]
    out_ref[...] = x2 + h2


def kernel(x, pos_w, pos_b, ln1_w, ln1_b, qkv_w, qkv_b, lepe_w, lepe_b,
           wo_w, wo_b, ln2_w, ln2_b, mlp_w1, mlp_b1, mlp_w2, mlp_b2):
    f32 = jnp.float32
    xh = x.transpose(0, 2, 3, 1)
    posw = pos_w.reshape(_D, 3, 3).transpose(1, 2, 0).reshape(9, _D)
    lepew = lepe_w.reshape(_D, 5, 5).transpose(1, 2, 0).reshape(25, _D)
    qkvwt = qkv_w.T.astype(jnp.bfloat16)
    wowt = wo_w.T.astype(jnp.bfloat16)
    w1t = mlp_w1.T.astype(jnp.bfloat16)
    w2t = mlp_w2.T.astype(jnp.bfloat16)

    full = lambda s: pl.BlockSpec(s, lambda n: tuple(0 for _ in s))
    x1, qkv, ridx = pl.pallas_call(
        _pre_body,
        grid=(_B,),
        in_specs=[
            pl.BlockSpec((1, _H, _W, _D), lambda n: (n, 0, 0, 0)),
            full((9, _D)), full((_D,)), full((_D,)), full((_D,)),
            full((_D, 2 * _QK + _D)), full((2 * _QK + _D,)),
        ],
        out_specs=[
            pl.BlockSpec((1, _H, _W, _D), lambda n: (n, 0, 0, 0)),
            pl.BlockSpec((1, _H, _W, 2 * _QK + _D), lambda n: (n, 0, 0, 0)),
            pl.BlockSpec((1, _P2, _TOPK), lambda n: (n, 0, 0)),
        ],
        out_shape=[
            jax.ShapeDtypeStruct((_B, _H, _W, _D), f32),
            jax.ShapeDtypeStruct((_B, _H, _W, 2 * _QK + _D), f32),
            jax.ShapeDtypeStruct((_B, _P2, _TOPK), jnp.int32),
        ],
    )(xh, posw, pos_b, ln1_w, ln1_b, qkvwt, qkv_b)

    attn = pl.pallas_call(
        _attn_body,
        grid_spec=pltpu.PrefetchScalarGridSpec(
            num_scalar_prefetch=1,
            grid=(_B, _NW),
            in_specs=[
                pl.BlockSpec((1, _H, _W, 2 * _QK + _D), lambda n, j, s: (n, 0, 0, 0)),
            ],
            out_specs=pl.BlockSpec((1, 8, _W, _D), lambda n, j, s: (n, j, 0, 0)),
            scratch_shapes=[pltpu.VMEM((_M * _W2, _QK), jnp.bfloat16)],
        ),
        out_shape=jax.ShapeDtypeStruct((_B, _H, _W, _D), f32),
    )(ridx.reshape(-1), qkv)

    x2 = pl.pallas_call(
        _post1_body,
        grid=(_B,),
        in_specs=[
            pl.BlockSpec((1, _H, _W, _D), lambda n: (n, 0, 0, 2)),
            pl.BlockSpec((1, _H, _W, _D), lambda n: (n, 0, 0, 0)),
            pl.BlockSpec((1, _H, _W, _D), lambda n: (n, 0, 0, 0)),
            full((25, _D)), full((_D,)), full((_D, _D)), full((_D,)),
        ],
        out_specs=[pl.BlockSpec((1, _H, _W, _D), lambda n: (n, 0, 0, 0))],
        out_shape=[jax.ShapeDtypeStruct((_B, _H, _W, _D), f32)],
    )(qkv, attn, x1, lepew, lepe_b, wowt, wo_b)[0]

    rows = _B * _H * _W
    tile = rows // 8
    out = pl.pallas_call(
        _post2_body,
        grid=(8,),
        in_specs=[
            pl.BlockSpec((tile, _D), lambda r: (r, 0)),
            full((_D,)), full((_D,)),
            full((_D, _HID)), full((_HID,)), full((_HID, _D)), full((_D,)),
        ],
        out_specs=pl.BlockSpec((tile, _D), lambda r: (r, 0)),
        out_shape=jax.ShapeDtypeStruct((rows, _D), f32),
    )(x2.reshape(rows, _D), ln2_w, ln2_b, w1t, mlp_b1, w2t, mlp_b2)

    return out.reshape(_B, _H, _W, _D).transpose(0, 3, 1, 2)
